# baseline (speedup 1.0000x reference)
import functools

import jax
import jax.numpy as jnp
from jax import lax
from jax.experimental import pallas as pl
from jax.experimental.pallas import tpu as pltpu
from jax.experimental.pallas import tpu_sc as plsc

VOCAB = 1000
VPAD = 1024
NREP = 64
VEXT = VOCAB + NREP
EMB = 128
B = 4096
L = 20
LPAD = 24
N = B * L

NC = 2
NS = 16
NW = NC * NS
HB = B // 2
BAT_W = HB // NW
NBUF = 4


def _table_body(emb_ref, wt_ref, b_ref, out_ref):
    g = jnp.dot(emb_ref[...], wt_ref[...], preferred_element_type=jnp.float32)
    g = g + b_ref[...]
    m = jnp.max(g, axis=1, keepdims=True)
    e = jnp.exp(g - m)
    lse = jnp.log(jnp.sum(e, axis=1, keepdims=True))
    out_ref[...] = g - (m + lse)


def _compute_table(embedding, fc1_w, fc1_b):
    wt_pad = jnp.pad(fc1_w.T, ((0, 0), (0, VPAD - VOCAB)))
    b_pad = jnp.pad(
        fc1_b.reshape(1, VOCAB), ((0, 0), (0, VPAD - VOCAB)),
        constant_values=-1e30,
    )
    return pl.pallas_call(
        _table_body,
        out_shape=jax.ShapeDtypeStruct((VOCAB, VPAD), jnp.float32),
    )(embedding, wt_pad, b_pad)


def _ids_body(x_ref, m_ref, out_ref):
    t = x_ref[...] * m_ref[...]
    pos = (lax.broadcasted_iota(jnp.int32, (B, LPAD), 0) * LPAD
           + lax.broadcasted_iota(jnp.int32, (B, LPAD), 1))
    repl = VOCAB + (pos & (NREP - 1))
    lpos = lax.broadcasted_iota(jnp.int32, (B, LPAD), 1)
    tp = jnp.pad(t, ((0, 0), (0, LPAD - L)))
    out_ref[...] = jnp.where((tp == 0) | (lpos >= L), repl, tp)


def _compute_ids(x, mask):
    return pl.pallas_call(
        _ids_body,
        out_shape=jax.ShapeDtypeStruct((B, LPAD), jnp.int32),
    )(x.astype(jnp.int32), mask.astype(jnp.int32))


def _gather_body(ids_hbm, tab_hbm, out_hbm, idv, bufs, gsems, ssems):
    wid = lax.axis_index("s") * NC + lax.axis_index("c")
    bbase = wid * BAT_W
    pltpu.sync_copy(ids_hbm.at[pl.ds(bbase, BAT_W)], idv)

    def g_start(j, b):
        pltpu.async_copy(tab_hbm.at[idv.at[j]], bufs[b], gsems[b])

    def g_wait(b):
        pltpu.make_async_copy(tab_hbm.at[idv.at[0]],
                              bufs[b], gsems[b]).wait()

    def s_start(j, b):
        pltpu.async_copy(bufs[b], out_hbm.at[bbase + j], ssems[b])

    def s_wait(b):
        pltpu.make_async_copy(bufs[b], out_hbm.at[bbase], ssems[b]).wait()

    g_start(0, 0)
    g_start(1, 1)

    def group(gi, carry):
        for b in range(NBUF):
            j = gi * NBUF + b
            jj = j + 2
            b2 = (b + 2) % NBUF

            @pl.when(jj < BAT_W)
            def _issue():
                @pl.when(j >= 2)
                def _free():
                    s_wait(b2)
                g_start(jj, b2)

            g_wait(b)
            s_start(j, b)
        return carry

    lax.fori_loop(0, BAT_W // NBUF, group, 0)
    for b in range(NBUF):
        s_wait(b)


_sc_gather = functools.partial(
    pl.kernel,
    out_type=jax.ShapeDtypeStruct((HB, LPAD, VPAD), jnp.float32),
    mesh=plsc.VectorSubcoreMesh(
        core_axis_name="c", subcore_axis_name="s", num_cores=NC, num_subcores=NS
    ),
    scratch_types=[
        pltpu.VMEM((BAT_W, LPAD), jnp.int32),
        [pltpu.VMEM((LPAD, VPAD), jnp.float32) for _ in range(NBUF)],
        [pltpu.SemaphoreType.DMA for _ in range(NBUF)],
        [pltpu.SemaphoreType.DMA for _ in range(NBUF)],
    ],
)(_gather_body)


def kernel(x, mask, embedding, fc1_w, fc1_b):
    table = _compute_table(embedding, fc1_w, fc1_b)
    ids = _compute_ids(x, mask)
    table_ext = jnp.concatenate(
        [table, jnp.broadcast_to(table[0:1], (NREP, VPAD))], axis=0)
    o0 = _sc_gather(ids[:HB], table_ext)
    o1 = _sc_gather(ids[HB:], table_ext)
    return jnp.concatenate(
        [o0[:, :L, :VOCAB], o1[:, :L, :VOCAB]], axis=0)


# speedup vs baseline: 1.7010x; 1.7010x over previous
import functools

import jax
import jax.numpy as jnp
from jax import lax
from jax.experimental import pallas as pl
from jax.experimental.pallas import tpu as pltpu
from jax.experimental.pallas import tpu_sc as plsc

VOCAB = 1000
VPAD = 1024
NREP = 344
VEXT = VOCAB + NREP
EMB = 128
B = 4096
L = 20
LPAD = 24
N = B * L

NC = 2
NS = 16
NW = NC * NS
BAT_W = B // NW
NBUF = 4


def _table_body(emb_ref, wt_ref, b_ref, out_ref):
    g = jnp.dot(emb_ref[...], wt_ref[...], preferred_element_type=jnp.float32)
    g = g + b_ref[...]
    m = jnp.max(g, axis=1, keepdims=True)
    e = jnp.exp(g - m)
    lse = jnp.log(jnp.sum(e, axis=1, keepdims=True))
    out_ref[...] = g - (m + lse)


def _compute_table(embedding, fc1_w, fc1_b):
    wt_pad = jnp.pad(fc1_w.T, ((0, 0), (0, VPAD - VOCAB)))
    b_pad = jnp.pad(
        fc1_b.reshape(1, VOCAB), ((0, 0), (0, VPAD - VOCAB)),
        constant_values=-1e30,
    )
    return pl.pallas_call(
        _table_body,
        out_shape=jax.ShapeDtypeStruct((VOCAB, VPAD), jnp.float32),
    )(embedding, wt_pad, b_pad)


def _ids_body(x_ref, m_ref, out_ref):
    t = x_ref[...] * m_ref[...]
    pos = (lax.broadcasted_iota(jnp.int32, (B, LPAD), 0) * LPAD
           + lax.broadcasted_iota(jnp.int32, (B, LPAD), 1))
    repl = VOCAB + lax.rem(pos, NREP)
    lpos = lax.broadcasted_iota(jnp.int32, (B, LPAD), 1)
    tp = jnp.pad(t, ((0, 0), (0, LPAD - L)))
    out_ref[...] = jnp.where((tp == 0) | (lpos >= L), repl, tp)


def _compute_ids(x, mask):
    return pl.pallas_call(
        _ids_body,
        out_shape=jax.ShapeDtypeStruct((B, LPAD), jnp.int32),
    )(x.astype(jnp.int32), mask.astype(jnp.int32))


def _gather_body(ids_hbm, tab_hbm, out_hbm, idv, bufs, gsems, ssems):
    wid = lax.axis_index("s") * NC + lax.axis_index("c")
    bbase = wid * BAT_W
    pltpu.sync_copy(ids_hbm.at[pl.ds(bbase, BAT_W)], idv)

    def g_start(j, b):
        pltpu.async_copy(tab_hbm.at[idv.at[j]], bufs[b], gsems[b])

    def g_wait(b):
        pltpu.make_async_copy(tab_hbm.at[idv.at[0]],
                              bufs[b], gsems[b]).wait()

    def s_start(j, b):
        pltpu.async_copy(bufs[b], out_hbm.at[bbase + j], ssems[b])

    def s_wait(b):
        pltpu.make_async_copy(bufs[b], out_hbm.at[bbase], ssems[b]).wait()

    g_start(0, 0)
    g_start(1, 1)

    def group(gi, carry):
        for b in range(NBUF):
            j = gi * NBUF + b
            jj = j + 2
            b2 = (b + 2) % NBUF

            @pl.when(jj < BAT_W)
            def _issue():
                @pl.when(j >= 2)
                def _free():
                    s_wait(b2)
                g_start(jj, b2)

            g_wait(b)
            s_start(j, b)
        return carry

    lax.fori_loop(0, BAT_W // NBUF, group, 0)
    for b in range(NBUF):
        s_wait(b)


_sc_gather = functools.partial(
    pl.kernel,
    out_type=jax.ShapeDtypeStruct((B, LPAD, VPAD), jnp.float32),
    mesh=plsc.VectorSubcoreMesh(
        core_axis_name="c", subcore_axis_name="s", num_cores=NC, num_subcores=NS
    ),
    scratch_types=[
        pltpu.VMEM((BAT_W, LPAD), jnp.int32),
        [pltpu.VMEM((LPAD, VPAD), jnp.float32) for _ in range(NBUF)],
        [pltpu.SemaphoreType.DMA for _ in range(NBUF)],
        [pltpu.SemaphoreType.DMA for _ in range(NBUF)],
    ],
)(_gather_body)


def kernel(x, mask, embedding, fc1_w, fc1_b):
    table = _compute_table(embedding, fc1_w, fc1_b)
    ids = _compute_ids(x, mask)
    table_ext = jnp.concatenate(
        [table, jnp.broadcast_to(table[0:1], (NREP, VPAD))], axis=0)
    return _sc_gather(ids, table_ext)[:, :L, :VOCAB]


# speedup vs baseline: 1.7375x; 1.0215x over previous
import functools

import jax
import jax.numpy as jnp
from jax import lax
from jax.experimental import pallas as pl
from jax.experimental.pallas import tpu as pltpu
from jax.experimental.pallas import tpu_sc as plsc

VOCAB = 1000
VPAD = 1024
NREP = 1048
VEXT = VOCAB + NREP
EMB = 128
B = 4096
L = 20
LPAD = 24
N = B * L

NC = 2
NS = 16
NW = NC * NS
BAT_W = B // NW
NBUF = 4


def _table_body(emb_ref, wt_ref, b_ref, out_ref):
    g = jnp.dot(emb_ref[...], wt_ref[...], preferred_element_type=jnp.float32)
    g = g + b_ref[...]
    m = jnp.max(g, axis=1, keepdims=True)
    e = jnp.exp(g - m)
    lse = jnp.log(jnp.sum(e, axis=1, keepdims=True))
    out_ref[...] = g - (m + lse)


def _compute_table(embedding, fc1_w, fc1_b):
    wt_pad = jnp.pad(fc1_w.T, ((0, 0), (0, VPAD - VOCAB)))
    b_pad = jnp.pad(
        fc1_b.reshape(1, VOCAB), ((0, 0), (0, VPAD - VOCAB)),
        constant_values=-1e30,
    )
    return pl.pallas_call(
        _table_body,
        out_shape=jax.ShapeDtypeStruct((VOCAB, VPAD), jnp.float32),
    )(embedding, wt_pad, b_pad)


def _ids_body(x_ref, m_ref, out_ref):
    t = x_ref[...] * m_ref[...]
    pos = (lax.broadcasted_iota(jnp.int32, (B, LPAD), 0) * LPAD
           + lax.broadcasted_iota(jnp.int32, (B, LPAD), 1))
    repl = VOCAB + lax.rem(pos, NREP)
    lpos = lax.broadcasted_iota(jnp.int32, (B, LPAD), 1)
    tp = jnp.pad(t, ((0, 0), (0, LPAD - L)))
    out_ref[...] = jnp.where((tp == 0) | (lpos >= L), repl, tp)


def _compute_ids(x, mask):
    return pl.pallas_call(
        _ids_body,
        out_shape=jax.ShapeDtypeStruct((B, LPAD), jnp.int32),
    )(x.astype(jnp.int32), mask.astype(jnp.int32))


def _gather_body(ids_hbm, tab_hbm, out_hbm, idv, bufs, gsems, ssems):
    wid = lax.axis_index("s") * NC + lax.axis_index("c")
    bbase = wid * BAT_W
    pltpu.sync_copy(ids_hbm.at[pl.ds(bbase, BAT_W)], idv)

    def g_start(j, b):
        pltpu.async_copy(tab_hbm.at[idv.at[j]], bufs[b], gsems[b])

    def g_wait(b):
        pltpu.make_async_copy(tab_hbm.at[idv.at[0]],
                              bufs[b], gsems[b]).wait()

    def s_start(j, b):
        pltpu.async_copy(bufs[b], out_hbm.at[bbase + j], ssems[b])

    def s_wait(b):
        pltpu.make_async_copy(bufs[b], out_hbm.at[bbase], ssems[b]).wait()

    g_start(0, 0)
    g_start(1, 1)

    def group(gi, carry):
        for b in range(NBUF):
            j = gi * NBUF + b
            jj = j + 2
            b2 = (b + 2) % NBUF

            @pl.when(jj < BAT_W)
            def _issue():
                @pl.when(j >= 2)
                def _free():
                    s_wait(b2)
                g_start(jj, b2)

            g_wait(b)
            s_start(j, b)
        return carry

    lax.fori_loop(0, BAT_W // NBUF, group, 0)
    for b in range(NBUF):
        s_wait(b)


_sc_gather = functools.partial(
    pl.kernel,
    out_type=jax.ShapeDtypeStruct((B, LPAD, VPAD), jnp.float32),
    mesh=plsc.VectorSubcoreMesh(
        core_axis_name="c", subcore_axis_name="s", num_cores=NC, num_subcores=NS
    ),
    scratch_types=[
        pltpu.VMEM((BAT_W, LPAD), jnp.int32),
        [pltpu.VMEM((LPAD, VPAD), jnp.float32) for _ in range(NBUF)],
        [pltpu.SemaphoreType.DMA for _ in range(NBUF)],
        [pltpu.SemaphoreType.DMA for _ in range(NBUF)],
    ],
)(_gather_body)


def kernel(x, mask, embedding, fc1_w, fc1_b):
    table = _compute_table(embedding, fc1_w, fc1_b)
    ids = _compute_ids(x, mask)
    table_ext = jnp.concatenate(
        [table, jnp.broadcast_to(table[0:1], (NREP, VPAD))], axis=0)
    return _sc_gather(ids, table_ext)[:, :L, :VOCAB]
